# Initial kernel scaffold; baseline (speedup 1.0000x reference)
import jax
import jax.numpy as jnp
from jax.experimental import pallas as pl


def kernel(severity_ids, weight):
    raise NotImplementedError("write your pallas kernel here")


# speedup vs baseline: 1.5598x; 1.5598x over previous
import functools

import jax
import jax.numpy as jnp
from jax import lax
from jax.experimental import pallas as pl
from jax.experimental.pallas import tpu as pltpu
from jax.experimental.pallas import tpu_sc as plsc

NUM_CLASSES = 1000000
EMBED_DIM = 32
BATCH = 16384
FIELDS = 26

NC = 2
NS = 16
NW = NC * NS
TOTAL = BATCH * FIELDS
PER_W = TOTAL // NW
IDX_B = 128
GPC = 8
CHUNK = IDX_B * GPC
N_CHUNKS = PER_W // CHUNK
N_GATHER = PER_W // IDX_B

assert PER_W * NW == TOTAL and CHUNK * N_CHUNKS == PER_W


def _make_gather():
    mesh = plsc.VectorSubcoreMesh(core_axis_name="c", subcore_axis_name="s")

    @functools.partial(
        pl.kernel,
        mesh=mesh,
        out_type=jax.ShapeDtypeStruct((TOTAL, EMBED_DIM), jnp.float32),
        scratch_types=[
            pltpu.VMEM((N_GATHER, IDX_B), jnp.int32),
            pltpu.VMEM((CHUNK, EMBED_DIM), jnp.float32),
            pltpu.SemaphoreType.DMA,
        ],
        compiler_params=pltpu.CompilerParams(use_tc_tiling_on_sc=False),
    )
    def gather_kernel(table_hbm, idx_hbm, out_hbm, idx_v, rows_v, sem):
        wid = lax.axis_index("s") * NC + lax.axis_index("c")
        pltpu.sync_copy(idx_hbm.at[wid], idx_v)

        def chunk_body(c, _):
            copies = [
                pltpu.async_copy(
                    table_hbm.at[idx_v.at[c * GPC + j]],
                    rows_v.at[pl.ds(j * IDX_B, IDX_B)],
                    sem,
                )
                for j in range(GPC)
            ]
            for cp in copies:
                cp.wait()
            base = wid * PER_W + c * CHUNK
            pltpu.sync_copy(rows_v, out_hbm.at[pl.ds(base, CHUNK)])
            return 0

        lax.fori_loop(0, N_CHUNKS, chunk_body, 0)

    return gather_kernel


_gather = _make_gather()


def kernel(severity_ids, weight):
    idx = severity_ids.reshape(NW, N_GATHER, IDX_B).astype(jnp.int32)
    out = _gather(weight, idx)
    return out.reshape(BATCH, FIELDS, EMBED_DIM)


# speedup vs baseline: 1.5714x; 1.0074x over previous
import functools

import jax
import jax.numpy as jnp
from jax import lax
from jax.experimental import pallas as pl
from jax.experimental.pallas import tpu as pltpu
from jax.experimental.pallas import tpu_sc as plsc

NUM_CLASSES = 1000000
EMBED_DIM = 32
BATCH = 16384
FIELDS = 26

NC = 2
NS = 16
NW = NC * NS
TOTAL = BATCH * FIELDS
PER_W = TOTAL // NW
IDX_B = 128
GPC = 2
CHUNK = IDX_B * GPC
N_CHUNKS = PER_W // CHUNK
N_GATHER = PER_W // IDX_B
NBUF = 4
N_MAIN = N_CHUNKS // NBUF - 1

assert PER_W * NW == TOTAL
assert CHUNK * N_CHUNKS == PER_W
assert N_CHUNKS % NBUF == 0


def _make_gather():
    mesh = plsc.VectorSubcoreMesh(core_axis_name="c", subcore_axis_name="s")

    @functools.partial(
        pl.kernel,
        mesh=mesh,
        out_type=jax.ShapeDtypeStruct((TOTAL, EMBED_DIM), jnp.float32),
        scratch_types=[
            pltpu.VMEM((N_GATHER, IDX_B), jnp.int32),
            pltpu.VMEM((NBUF, CHUNK, EMBED_DIM), jnp.float32),
        ]
        + [pltpu.SemaphoreType.DMA] * (2 * NBUF),
        compiler_params=pltpu.CompilerParams(use_tc_tiling_on_sc=False),
    )
    def gather_kernel(table_hbm, idx_hbm, out_hbm, idx_v, rows_v, *sems):
        gsem = sems[:NBUF]
        osem = sems[NBUF:]
        wid = lax.axis_index("s") * NC + lax.axis_index("c")
        pltpu.sync_copy(idx_hbm.at[wid], idx_v)

        def start_gather(c, b):
            for j in range(GPC):
                pltpu.async_copy(
                    table_hbm.at[idx_v.at[c * GPC + j]],
                    rows_v.at[b].at[pl.ds(j * IDX_B, IDX_B)],
                    gsem[b],
                )

        def wait_gather(b):
            for j in range(GPC):
                pltpu.make_async_copy(
                    table_hbm.at[idx_v.at[j]],
                    rows_v.at[b].at[pl.ds(j * IDX_B, IDX_B)],
                    gsem[b],
                ).wait()

        def start_out(c, b):
            pltpu.async_copy(
                rows_v.at[b],
                out_hbm.at[pl.ds(wid * PER_W + c * CHUNK, CHUNK)],
                osem[b],
            )

        def wait_out(b):
            pltpu.make_async_copy(
                rows_v.at[b],
                out_hbm.at[pl.ds(wid * PER_W, CHUNK)],
                osem[b],
            ).wait()

        for b in range(NBUF):
            start_gather(jnp.int32(b), b)

        def body(g, _):
            for b in range(NBUF):
                c = g * NBUF + b
                wait_gather(b)
                start_out(c, b)
            for b in range(NBUF):
                c_next = (g + 1) * NBUF + b
                wait_out(b)
                start_gather(c_next, b)
            return 0

        lax.fori_loop(0, N_MAIN, body, 0)

        for b in range(NBUF):
            c = N_MAIN * NBUF + b
            wait_gather(b)
            start_out(jnp.int32(c), b)
        for b in range(NBUF):
            wait_out(b)

    return gather_kernel


_gather = _make_gather()


def kernel(severity_ids, weight):
    idx = severity_ids.reshape(NW, N_GATHER, IDX_B).astype(jnp.int32)
    out = _gather(weight, idx)
    return out.reshape(BATCH, FIELDS, EMBED_DIM)
